# baseline (speedup 1.0000x reference)
import functools

import jax
import jax.numpy as jnp
from jax import lax
from jax.experimental import pallas as pl
from jax.experimental.pallas import tpu as pltpu
from jax.experimental.pallas import tpu_sc as plsc

NUM_ENTITIES = 1000000
NUM_RELATIONS = 1000
EMBED_DIM = 64
BATCH = 16384

NC = 2
NS = 16
NW = NC * NS
BPW = BATCH // NW
LANES = 16
NSEM = 4
WAVE = 256
NWAVE = BPW // WAVE
RCHUNK = 200


def _make_sc_lookup():
  mesh = plsc.VectorSubcoreMesh(core_axis_name="c", subcore_axis_name="s")

  @functools.partial(
      pl.kernel,
      mesh=mesh,
      compiler_params=pltpu.CompilerParams(needs_layout_passes=False),
      out_type=(
          jax.ShapeDtypeStruct((BATCH, EMBED_DIM), jnp.float32),
          jax.ShapeDtypeStruct((BATCH, EMBED_DIM), jnp.float32),
      ),
      scratch_types=[
          pltpu.VMEM((BPW,), jnp.int32),
          pltpu.VMEM((BPW,), jnp.int32),
          pltpu.VMEM((WAVE, EMBED_DIM), jnp.float32),
          pltpu.VMEM((WAVE, EMBED_DIM), jnp.float32),
          pltpu.VMEM((RCHUNK, EMBED_DIM), jnp.float32),
          [pltpu.SemaphoreType.DMA] * NSEM,
      ],
  )
  def lookup(heads_hbm, rels_hbm, ent_hbm, rel_hbm, out_h, out_r,
             hidx_v, ridx_v, hrows, rrows, rtabc, hsems):
    wid = lax.axis_index("s") * NC + lax.axis_index("c")
    base = wid * BPW
    pltpu.sync_copy(heads_hbm.at[wid], hidx_v)
    pltpu.sync_copy(rels_hbm.at[wid], ridx_v)

    def fire(idx_v, table, rows, sems, wbase):
      def body(g, _):
        gb = wbase + g * LANES
        vec = idx_v[pl.ds(gb, LANES)]
        for j in range(LANES):
          pltpu.async_copy(table.at[pl.ds(vec[j], 1)],
                           rows.at[pl.ds(g * LANES + j, 1)],
                           sems[j % NSEM])
        return _
      lax.fori_loop(0, WAVE // LANES, body, 0)

    def drain(table, rows, sems):
      def body(i, _):
        for q in range(NSEM):
          pltpu.make_async_copy(table.at[pl.ds(0, 1)],
                                rows.at[pl.ds(i, 1)], sems[q]).wait()
        return _
      lax.fori_loop(0, WAVE // NSEM, body, 0)

    def relations(wbase):
      for q in range(NUM_RELATIONS // RCHUNK):
        lo = q * RCHUNK
        pltpu.sync_copy(rel_hbm.at[pl.ds(lo, RCHUNK)], rtabc)

        def body(g, carry, lo=lo):
          vec = ridx_v[pl.ds(wbase + g * LANES, LANES)]
          for j in range(LANES):
            r = vec[j]
            t = g * LANES + j

            @pl.when(jnp.logical_and(r >= lo, r < lo + RCHUNK))
            def _copy_row():
              for k in range(EMBED_DIM // LANES):
                rrows[t, pl.ds(k * LANES, LANES)] = (
                    rtabc[r - lo, pl.ds(k * LANES, LANES)])
          return carry

        lax.fori_loop(0, WAVE // LANES, body, 0)

    for w in range(NWAVE):
      wb = w * WAVE
      fire(hidx_v, ent_hbm, hrows, hsems, wb)
      relations(wb)
      pltpu.sync_copy(rrows, out_r.at[pl.ds(base + wb, WAVE)])
      drain(ent_hbm, hrows, hsems)
      pltpu.sync_copy(hrows, out_h.at[pl.ds(base + wb, WAVE)])

  return lookup


_lookup = _make_sc_lookup()


@jax.jit
def kernel(heads, relations, entity_table, relation_table):
  heads_r = heads.astype(jnp.int32).reshape(NW, BPW)
  rels_r = relations.astype(jnp.int32).reshape(NW, BPW)
  return _lookup(heads_r, rels_r, entity_table, relation_table)


# speedup vs baseline: 1.1459x; 1.1459x over previous
import functools

import jax
import jax.numpy as jnp
from jax import lax
from jax.experimental import pallas as pl
from jax.experimental.pallas import tpu as pltpu
from jax.experimental.pallas import tpu_sc as plsc

NUM_ENTITIES = 1000000
NUM_RELATIONS = 1000
EMBED_DIM = 64
BATCH = 16384

NC = 2
NS = 16
NW = NC * NS
BPW = BATCH // NW
LANES = 16
NSEM = 4
WAVE = 256
NWAVE = BPW // WAVE


def _make_sc_lookup():
  mesh = plsc.VectorSubcoreMesh(core_axis_name="c", subcore_axis_name="s")

  @functools.partial(
      pl.kernel,
      mesh=mesh,
      compiler_params=pltpu.CompilerParams(needs_layout_passes=False),
      out_type=(
          jax.ShapeDtypeStruct((BATCH, EMBED_DIM), jnp.float32),
          jax.ShapeDtypeStruct((BATCH, EMBED_DIM), jnp.float32),
      ),
      scratch_types=[
          pltpu.VMEM((BPW,), jnp.int32),
          pltpu.VMEM((BPW,), jnp.int32),
          pltpu.VMEM((WAVE, EMBED_DIM), jnp.float32),
          pltpu.VMEM((WAVE, EMBED_DIM), jnp.float32),
          [pltpu.SemaphoreType.DMA] * NSEM,
          [pltpu.SemaphoreType.DMA] * NSEM,
      ],
  )
  def lookup(heads_hbm, rels_hbm, ent_hbm, rel_hbm, out_h, out_r,
             hidx_v, ridx_v, hrows, rrows, hsems, rsems):
    wid = lax.axis_index("s") * NC + lax.axis_index("c")
    base = wid * BPW
    pltpu.sync_copy(heads_hbm.at[wid], hidx_v)
    pltpu.sync_copy(rels_hbm.at[wid], ridx_v)

    def fire(idx_v, table, rows, sems, wbase):
      def body(g, _):
        gb = wbase + g * LANES
        vec = idx_v[pl.ds(gb, LANES)]
        for j in range(LANES):
          pltpu.async_copy(table.at[pl.ds(vec[j], 1)],
                           rows.at[pl.ds(g * LANES + j, 1)],
                           sems[j % NSEM])
        return _
      lax.fori_loop(0, WAVE // LANES, body, 0)

    def drain(table, rows, sems):
      def body(i, _):
        for q in range(NSEM):
          pltpu.make_async_copy(table.at[pl.ds(0, 1)],
                                rows.at[pl.ds(i, 1)], sems[q]).wait()
        return _
      lax.fori_loop(0, WAVE // NSEM, body, 0)

    for w in range(NWAVE):
      wb = w * WAVE
      fire(hidx_v, ent_hbm, hrows, hsems, wb)
      fire(ridx_v, rel_hbm, rrows, rsems, wb)
      drain(ent_hbm, hrows, hsems)
      pltpu.sync_copy(hrows, out_h.at[pl.ds(base + wb, WAVE)])
      drain(rel_hbm, rrows, rsems)
      pltpu.sync_copy(rrows, out_r.at[pl.ds(base + wb, WAVE)])

  return lookup


_lookup = _make_sc_lookup()


@jax.jit
def kernel(heads, relations, entity_table, relation_table):
  heads_r = heads.astype(jnp.int32).reshape(NW, BPW)
  rels_r = relations.astype(jnp.int32).reshape(NW, BPW)
  return _lookup(heads_r, rels_r, entity_table, relation_table)
